# baseline (speedup 1.0000x reference)
import jax
import jax.numpy as jnp
from jax import lax
from jax.experimental import pallas as pl
from jax.experimental.pallas import tpu as pltpu
from jax.experimental.pallas import tpu_sc as plsc

NC = 2
NS = 16
L = 16


def _relayout(V, E, BK=8192):
    def body(t_ref, o_ref):
        eye = jnp.eye(E, dtype=jnp.float32)
        t = jax.lax.dot_general(t_ref[...], eye, (((0,), (0,)), ((), ())),
                                preferred_element_type=jnp.float32)
        o_ref[:, 0:E] = t[0:BK // 2]
        o_ref[:, E:2 * E] = t[BK // 2:BK]

    grid = (V + BK - 1) // BK

    return pl.pallas_call(
        body,
        grid=(grid,),
        in_specs=[pl.BlockSpec((E, BK), lambda i: (0, i))],
        out_specs=pl.BlockSpec((BK // 2, 2 * E), lambda i: (i, 0)),
        out_shape=jax.ShapeDtypeStruct((grid * (BK // 2), 2 * E),
                                       jnp.float32),
        compiler_params=pltpu.CompilerParams(
            dimension_semantics=("arbitrary",)),
    )


def _pooled_lookup(B, S, V2, E):
    NW = NC * NS
    BPW = B // NW
    SA = ((S // 2 + 7) // 8) * 8
    SB = S - SA
    NCH = E // L
    HALVES = ((0, SA), (SA, SB))

    mesh = plsc.VectorSubcoreMesh(core_axis_name="c", subcore_axis_name="s")

    def body(x_hbm, tbl_hbm, out_hbm, idx_v, rows0, rows1, out_v, sem0, sem1):
        wid = lax.axis_index("s") * NC + lax.axis_index("c")

        pltpu.sync_copy(x_hbm.at[pl.ds(wid * (2 * BPW), 2 * BPW)], idx_v)

        def fire(b, rows_ref, sem):
            for h, (off, n) in enumerate(HALVES):
                src = tbl_hbm.at[idx_v.at[2 * b + h, pl.ds(0, n)]]
                pltpu.async_copy(src, rows_ref.at[pl.ds(off, n)], sem)

        def drain(rows_ref, sem):
            for h, (off, n) in enumerate(HALVES):
                pltpu.make_async_copy(
                    tbl_hbm.at[idx_v.at[0, pl.ds(0, n)]],
                    rows_ref.at[pl.ds(off, n)], sem).wait()

        def consume(rows_ref, b):
            def accum(i, accs):
                return tuple(accs[j] + rows_ref[i, pl.ds(j * L, L)]
                             for j in range(NCH))
            init = tuple(jnp.zeros((L,), jnp.float32) for _ in range(NCH))
            accs = lax.fori_loop(0, S, accum, init)
            inv = jnp.float32(1.0 / S)
            for j in range(NCH):
                out_v[b, pl.ds(j * L, L)] = accs[j] * inv

        fire(0, rows0, sem0)

        def outer(bb, carry):
            b0 = 2 * bb
            fire(b0 + 1, rows1, sem1)
            drain(rows0, sem0)
            consume(rows0, b0)

            @pl.when(bb < BPW // 2 - 1)
            def _():
                fire(b0 + 2, rows0, sem0)

            drain(rows1, sem1)
            consume(rows1, b0 + 1)
            return carry

        lax.fori_loop(0, BPW // 2, outer, 0)
        pltpu.sync_copy(out_v, out_hbm.at[pl.ds(wid * BPW, BPW)])

    return pl.kernel(
        body,
        out_type=jax.ShapeDtypeStruct((B, E), jnp.float32),
        mesh=mesh,
        compiler_params=pltpu.CompilerParams(use_tc_tiling_on_sc=False),
        scratch_types=[
            pltpu.VMEM((2 * BPW, SA), jnp.int32),
            pltpu.VMEM((S, E), jnp.float32),
            pltpu.VMEM((S, E), jnp.float32),
            pltpu.VMEM((BPW, E), jnp.float32),
            pltpu.SemaphoreType.DMA,
            pltpu.SemaphoreType.DMA,
        ],
    )


def kernel(x, embedding_weight):
    B, S = x.shape
    V, E = embedding_weight.shape
    SA = ((S // 2 + 7) // 8) * 8
    BK = 8192
    xr = (x & -BK) + ((x & (BK // 2 - 1)) << 1) + ((x & (BK - 1)) >> 12)
    a = xr[:, :SA]
    bseg = jnp.pad(xr[:, SA:], ((0, 0), (0, 2 * SA - S)))
    x2 = jnp.stack([a, bseg], axis=1).reshape(2 * B, SA)
    tableC = _relayout(V, E, BK)(embedding_weight.T)
    V2 = tableC.shape[0] * 2
    tableL = tableC.reshape(V2, E)
    return _pooled_lookup(B, S, V2, E)(x2, tableL)


# speedup vs baseline: 1.0869x; 1.0869x over previous
import jax
import jax.numpy as jnp
from jax import lax
from jax.experimental import pallas as pl
from jax.experimental.pallas import tpu as pltpu
from jax.experimental.pallas import tpu_sc as plsc

NC = 2
NS = 16
L = 16


def _relayout(V, E, BK=8192):
    def body(t_ref, o_ref):
        t = jnp.swapaxes(t_ref[...], 0, 1)
        o_ref[:, 0:E] = t[0:BK // 2]
        o_ref[:, E:2 * E] = t[BK // 2:BK]

    grid = (V + BK - 1) // BK

    return pl.pallas_call(
        body,
        grid=(grid,),
        in_specs=[pl.BlockSpec((E, BK), lambda i: (0, i))],
        out_specs=pl.BlockSpec((BK // 2, 2 * E), lambda i: (i, 0)),
        out_shape=jax.ShapeDtypeStruct((grid * (BK // 2), 2 * E),
                                       jnp.float32),
        compiler_params=pltpu.CompilerParams(
            dimension_semantics=("arbitrary",)),
    )


def _pooled_lookup(B, S, V2, E):
    NW = NC * NS
    BPW = B // NW
    SA = ((S // 2 + 7) // 8) * 8
    SB = S - SA
    NCH = E // L
    HALVES = ((0, SA), (SA, SB))

    mesh = plsc.VectorSubcoreMesh(core_axis_name="c", subcore_axis_name="s")

    def body(x_hbm, tbl_hbm, out_hbm, idx_v, rows0, rows1, out_v, sem0, sem1):
        wid = lax.axis_index("s") * NC + lax.axis_index("c")

        pltpu.sync_copy(x_hbm.at[pl.ds(wid * (2 * BPW), 2 * BPW)], idx_v)

        def fire(b, rows_ref, sem):
            for h, (off, n) in enumerate(HALVES):
                src = tbl_hbm.at[idx_v.at[2 * b + h, pl.ds(0, n)]]
                pltpu.async_copy(src, rows_ref.at[pl.ds(off, n)], sem)

        def drain(rows_ref, sem):
            for h, (off, n) in enumerate(HALVES):
                pltpu.make_async_copy(
                    tbl_hbm.at[idx_v.at[0, pl.ds(0, n)]],
                    rows_ref.at[pl.ds(off, n)], sem).wait()

        def consume(rows_ref, b):
            def accum(i, accs):
                return tuple(accs[j] + rows_ref[i, pl.ds(j * L, L)]
                             for j in range(NCH))
            init = tuple(jnp.zeros((L,), jnp.float32) for _ in range(NCH))
            accs = lax.fori_loop(0, S, accum, init)
            inv = jnp.float32(1.0 / S)
            for j in range(NCH):
                out_v[b, pl.ds(j * L, L)] = accs[j] * inv

        fire(0, rows0, sem0)

        def outer(bb, carry):
            b0 = 2 * bb
            fire(b0 + 1, rows1, sem1)
            drain(rows0, sem0)
            consume(rows0, b0)

            @pl.when(bb < BPW // 2 - 1)
            def _():
                fire(b0 + 2, rows0, sem0)

            drain(rows1, sem1)
            consume(rows1, b0 + 1)
            return carry

        lax.fori_loop(0, BPW // 2, outer, 0)
        pltpu.sync_copy(out_v, out_hbm.at[pl.ds(wid * BPW, BPW)])

    return pl.kernel(
        body,
        out_type=jax.ShapeDtypeStruct((B, E), jnp.float32),
        mesh=mesh,
        compiler_params=pltpu.CompilerParams(use_tc_tiling_on_sc=False),
        scratch_types=[
            pltpu.VMEM((2 * BPW, SA), jnp.int32),
            pltpu.VMEM((S, E), jnp.float32),
            pltpu.VMEM((S, E), jnp.float32),
            pltpu.VMEM((BPW, E), jnp.float32),
            pltpu.SemaphoreType.DMA,
            pltpu.SemaphoreType.DMA,
        ],
    )


def kernel(x, embedding_weight):
    B, S = x.shape
    V, E = embedding_weight.shape
    SA = ((S // 2 + 7) // 8) * 8
    BK = 16384
    xr = (x & -BK) + ((x & (BK // 2 - 1)) << 1) + ((x & (BK - 1)) >> 13)
    a = xr[:, :SA]
    bseg = jnp.pad(xr[:, SA:], ((0, 0), (0, 2 * SA - S)))
    x2 = jnp.stack([a, bseg], axis=1).reshape(2 * B, SA)
    tableC = _relayout(V, E, BK)(embedding_weight.T)
    V2 = tableC.shape[0] * 2
    tableL = tableC.reshape(V2, E)
    return _pooled_lookup(B, S, V2, E)(x2, tableL)


# speedup vs baseline: 1.1477x; 1.0559x over previous
import jax
import jax.numpy as jnp
from jax import lax
from jax.experimental import pallas as pl
from jax.experimental.pallas import tpu as pltpu
from jax.experimental.pallas import tpu_sc as plsc

NC = 2
NS = 16
L = 16


def _relayout(V, E, BK=8192):
    def body(t_ref, o_ref):
        t = jnp.swapaxes(t_ref[...], 0, 1)
        o_ref[:, 0:E] = t[0:BK // 2]
        o_ref[:, E:2 * E] = t[BK // 2:BK]

    grid = (V + BK - 1) // BK

    return pl.pallas_call(
        body,
        grid=(grid,),
        in_specs=[pl.BlockSpec((E, BK), lambda i: (0, i))],
        out_specs=pl.BlockSpec((BK // 2, 2 * E), lambda i: (i, 0)),
        out_shape=jax.ShapeDtypeStruct((grid * (BK // 2), 2 * E),
                                       jnp.float32),
        compiler_params=pltpu.CompilerParams(
            dimension_semantics=("arbitrary",)),
    )


def _pooled_lookup(B, S, V2, E):
    NW = NC * NS
    BPW = B // NW
    SA = ((S // 2 + 7) // 8) * 8
    SB = S - SA
    NCH = E // L
    HALVES = ((0, SA), (SA, SB))

    mesh = plsc.VectorSubcoreMesh(core_axis_name="c", subcore_axis_name="s")

    def body(x_hbm, tbl_hbm, out_hbm, idx_v, rows0, rows1, out_v, sem0, sem1):
        wid = lax.axis_index("s") * NC + lax.axis_index("c")

        pltpu.sync_copy(x_hbm.at[pl.ds(wid * (2 * BPW), 2 * BPW)], idx_v)

        def fire(b, rows_ref, sem):
            for h, (off, n) in enumerate(HALVES):
                src = tbl_hbm.at[idx_v.at[2 * b + h, pl.ds(0, n)]]
                pltpu.async_copy(src, rows_ref.at[pl.ds(off, n)], sem)

        def drain(rows_ref, sem):
            for h, (off, n) in enumerate(HALVES):
                pltpu.make_async_copy(
                    tbl_hbm.at[idx_v.at[0, pl.ds(0, n)]],
                    rows_ref.at[pl.ds(off, n)], sem).wait()

        def consume(rows_ref, b):
            def accum(i, accs):
                return tuple(accs[j] + rows_ref[i, pl.ds(j * L, L)]
                             for j in range(NCH))
            init = tuple(jnp.zeros((L,), jnp.float32) for _ in range(NCH))
            accs = lax.fori_loop(0, S, accum, init)
            inv = jnp.float32(1.0 / S)
            for j in range(NCH):
                out_v[b, pl.ds(j * L, L)] = accs[j] * inv

        fire(0, rows0, sem0)

        def outer(bb, carry):
            b0 = 2 * bb
            fire(b0 + 1, rows1, sem1)
            drain(rows0, sem0)
            consume(rows0, b0)

            @pl.when(bb < BPW // 2 - 1)
            def _():
                fire(b0 + 2, rows0, sem0)

            drain(rows1, sem1)
            consume(rows1, b0 + 1)
            return carry

        lax.fori_loop(0, BPW // 2, outer, 0)
        pltpu.sync_copy(out_v, out_hbm.at[pl.ds(wid * BPW, BPW)])

    return pl.kernel(
        body,
        out_type=jax.ShapeDtypeStruct((B, E), jnp.float32),
        mesh=mesh,
        compiler_params=pltpu.CompilerParams(use_tc_tiling_on_sc=False),
        scratch_types=[
            pltpu.VMEM((2 * BPW, SA), jnp.int32),
            pltpu.VMEM((S, E), jnp.float32),
            pltpu.VMEM((S, E), jnp.float32),
            pltpu.VMEM((BPW, E), jnp.float32),
            pltpu.SemaphoreType.DMA,
            pltpu.SemaphoreType.DMA,
        ],
    )


def kernel(x, embedding_weight):
    B, S = x.shape
    V, E = embedding_weight.shape
    SA = ((S // 2 + 7) // 8) * 8
    BK = 32768
    xr = (x & -BK) + ((x & (BK // 2 - 1)) << 1) + ((x & (BK - 1)) >> 14)
    a = xr[:, :SA]
    bseg = jnp.pad(xr[:, SA:], ((0, 0), (0, 2 * SA - S)))
    x2 = jnp.stack([a, bseg], axis=1).reshape(2 * B, SA)
    tableC = _relayout(V, E, BK)(embedding_weight.T)
    V2 = tableC.shape[0] * 2
    tableL = tableC.reshape(V2, E)
    return _pooled_lookup(B, S, V2, E)(x2, tableL)
